# baseline (device time: 154699 ns/iter reference)
import jax
import jax.numpy as jnp
from jax import lax
from jax.experimental import pallas as pl
from jax.experimental.pallas import tpu as pltpu

N_DEV = 4
BM = 1024
BK = 1024
BN = 1024
N_TOTAL = 8192
N_TILES = N_TOTAL // BN
N_PAIR = 2


def kernel(x, w_mat):
    x = x.astype(jnp.bfloat16)
    m_full, k_shard = x.shape
    k_full, n_total = w_mat.shape

    my = lax.axis_index("i")
    pair_order = jnp.stack([my // 2, 1 - my // 2]).astype(jnp.int32)

    def body(ord_ref, x_ref, w_ref, out_ref, comm_ref, acc_ref,
             send_sems, recv_sems):
        p = pl.program_id(0)
        n = pl.program_id(1)
        me = lax.axis_index("i")
        partner = jnp.bitwise_xor(me, 1)
        other_nbr = jnp.where(me % 2 == 0, (me + 3) % N_DEV, (me + 1) % N_DEV)
        diag = (me + 2) % N_DEV

        @pl.when(jnp.logical_and(p == 0, n == 0))
        def _():
            barrier = pltpu.get_barrier_semaphore()
            for off in (1, 2, 3):
                pl.semaphore_signal(
                    barrier, inc=1,
                    device_id=((me + off) % N_DEV,),
                    device_id_type=pl.DeviceIdType.MESH,
                )
            pl.semaphore_wait(barrier, N_DEV - 1)

            comm_ref[:, pl.ds(me * BK, BK)] = x_ref[pl.ds(me * BM, BM), :]

            for d in (partner, other_nbr, diag):
                rdma = pltpu.make_async_remote_copy(
                    src_ref=x_ref.at[pl.ds(d * BM, BM), :],
                    dst_ref=comm_ref.at[:, pl.ds(me * BK, BK)],
                    send_sem=send_sems.at[d],
                    recv_sem=recv_sems.at[me],
                    device_id=(d,),
                    device_id_type=pl.DeviceIdType.MESH,
                )
                rdma.start()

        def wait_recv_from(s):
            recv = pltpu.make_async_remote_copy(
                src_ref=comm_ref.at[:, pl.ds(s * BK, BK)],
                dst_ref=comm_ref.at[:, pl.ds(s * BK, BK)],
                send_sem=send_sems.at[s],
                recv_sem=recv_sems.at[s],
                device_id=(me,),
                device_id_type=pl.DeviceIdType.MESH,
            )
            recv.wait_recv()

        @pl.when(jnp.logical_and(p == 0, n == 0))
        def _():
            wait_recv_from(partner)

        @pl.when(jnp.logical_and(p == 1, n == 0))
        def _():
            wait_recv_from(other_nbr)
            wait_recv_from(diag)

        pair = ord_ref[p]
        partial = lax.dot_general(
            comm_ref[:, pl.ds(pair * (N_PAIR * BK), N_PAIR * BK)],
            w_ref[...],
            (((1,), (0,)), ((), ())),
            preferred_element_type=jnp.float32,
        )
        nsl = pl.ds(n * BN, BN)

        @pl.when(p == 0)
        def _():
            acc_ref[:, nsl] = partial.astype(jnp.bfloat16)

        @pl.when(p == 1)
        def _():
            out_ref[...] = jnp.maximum(
                acc_ref[:, nsl].astype(jnp.float32) + partial, 0.0
            )

        @pl.when(jnp.logical_and(p == 1, n == N_TILES - 1))
        def _():
            for d in (partner, other_nbr, diag):
                send = pltpu.make_async_remote_copy(
                    src_ref=x_ref.at[pl.ds(d * BM, BM), :],
                    dst_ref=comm_ref.at[:, pl.ds(me * BK, BK)],
                    send_sem=send_sems.at[d],
                    recv_sem=recv_sems.at[me],
                    device_id=(d,),
                    device_id_type=pl.DeviceIdType.MESH,
                )
                send.wait_send()

    grid_spec = pltpu.PrefetchScalarGridSpec(
        num_scalar_prefetch=1,
        grid=(N_PAIR, N_TILES),
        in_specs=[
            pl.BlockSpec((m_full, k_shard), lambda p, n, o: (0, 0)),
            pl.BlockSpec((N_PAIR * BK, BN), lambda p, n, o: (o[p], n)),
        ],
        out_specs=pl.BlockSpec((BM, BN), lambda p, n, o: (0, n)),
        scratch_shapes=[
            pltpu.VMEM((BM, N_DEV * BK), jnp.bfloat16),
            pltpu.VMEM((BM, N_TOTAL), jnp.bfloat16),
            pltpu.SemaphoreType.DMA((N_DEV,)),
            pltpu.SemaphoreType.DMA((N_DEV,)),
        ],
    )
    return pl.pallas_call(
        body,
        grid_spec=grid_spec,
        out_shape=jax.ShapeDtypeStruct((BM, n_total), jnp.float32),
        compiler_params=pltpu.CompilerParams(
            dimension_semantics=("arbitrary", "arbitrary"),
            collective_id=0,
            vmem_limit_bytes=64 * 1024 * 1024,
        ),
    )(pair_order, x, w_mat)


# device time: 129510 ns/iter; 1.1945x vs baseline; 1.1945x over previous
import jax
import jax.numpy as jnp
from jax import lax
from jax.experimental import pallas as pl
from jax.experimental.pallas import tpu as pltpu

N_DEV = 4
BM = 1024
BK = 1024
BN = 2048
N_TOTAL = 8192
N_TILES = N_TOTAL // BN


def kernel(x, w_mat):
    x = x.astype(jnp.bfloat16)
    m_full, k_shard = x.shape
    k_full, n_total = w_mat.shape

    my = lax.axis_index("i")
    order = jnp.stack(
        [my, (my + 1) % N_DEV, (my + 3) % N_DEV, (my + 2) % N_DEV]
    ).astype(jnp.int32)

    def body(ord_ref, x_ref, w_ref, out_ref, comm_ref, acc_ref,
             send_sems, recv_sems):
        jj = pl.program_id(0)
        n = pl.program_id(1)
        me = lax.axis_index("i")
        src = ord_ref[jj]

        @pl.when(jnp.logical_and(jj == 0, n == 0))
        def _():
            barrier = pltpu.get_barrier_semaphore()
            for off in (1, 2, 3):
                pl.semaphore_signal(
                    barrier, inc=1,
                    device_id=((me + off) % N_DEV,),
                    device_id_type=pl.DeviceIdType.MESH,
                )
            pl.semaphore_wait(barrier, N_DEV - 1)

            comm_ref[me] = x_ref[pl.ds(me * BM, BM), :]

            for off in (3, 1, 2):
                d = (me + off) % N_DEV
                rdma = pltpu.make_async_remote_copy(
                    src_ref=x_ref.at[pl.ds(d * BM, BM), :],
                    dst_ref=comm_ref.at[me],
                    send_sem=send_sems.at[d],
                    recv_sem=recv_sems.at[me],
                    device_id=(d,),
                    device_id_type=pl.DeviceIdType.MESH,
                )
                rdma.start()

        @pl.when(jnp.logical_and(jj > 0, n == 0))
        def _():
            recv = pltpu.make_async_remote_copy(
                src_ref=comm_ref.at[src],
                dst_ref=comm_ref.at[src],
                send_sem=send_sems.at[src],
                recv_sem=recv_sems.at[src],
                device_id=(me,),
                device_id_type=pl.DeviceIdType.MESH,
            )
            recv.wait_recv()

        partial = lax.dot_general(
            comm_ref[src], w_ref[...],
            (((1,), (0,)), ((), ())),
            preferred_element_type=jnp.float32,
        )
        nsl = pl.ds(n * BN, BN)

        @pl.when(jj == 0)
        def _():
            acc_ref[:, nsl] = partial.astype(jnp.bfloat16)

        @pl.when(jnp.logical_and(jj > 0, jj < N_DEV - 1))
        def _():
            acc_ref[:, nsl] = (
                acc_ref[:, nsl].astype(jnp.float32) + partial
            ).astype(jnp.bfloat16)

        @pl.when(jj == N_DEV - 1)
        def _():
            out_ref[...] = jnp.maximum(
                acc_ref[:, nsl].astype(jnp.float32) + partial, 0.0
            ).astype(jnp.bfloat16)

        @pl.when(jnp.logical_and(jj == N_DEV - 1, n == N_TILES - 1))
        def _():
            for off in (3, 1, 2):
                d = (me + off) % N_DEV
                send = pltpu.make_async_remote_copy(
                    src_ref=x_ref.at[pl.ds(d * BM, BM), :],
                    dst_ref=comm_ref.at[me],
                    send_sem=send_sems.at[d],
                    recv_sem=recv_sems.at[me],
                    device_id=(d,),
                    device_id_type=pl.DeviceIdType.MESH,
                )
                send.wait_send()

    grid_spec = pltpu.PrefetchScalarGridSpec(
        num_scalar_prefetch=1,
        grid=(N_DEV, N_TILES),
        in_specs=[
            pl.BlockSpec((m_full, k_shard), lambda jj, n, o: (0, 0)),
            pl.BlockSpec((BK, BN), lambda jj, n, o: (o[jj], n)),
        ],
        out_specs=pl.BlockSpec((BM, BN), lambda jj, n, o: (0, n)),
        scratch_shapes=[
            pltpu.VMEM((N_DEV, BM, BK), jnp.bfloat16),
            pltpu.VMEM((BM, N_TOTAL), jnp.bfloat16),
            pltpu.SemaphoreType.DMA((N_DEV,)),
            pltpu.SemaphoreType.DMA((N_DEV,)),
        ],
    )
    return pl.pallas_call(
        body,
        grid_spec=grid_spec,
        out_shape=jax.ShapeDtypeStruct((BM, n_total), jnp.bfloat16),
        compiler_params=pltpu.CompilerParams(
            dimension_semantics=("arbitrary", "arbitrary"),
            collective_id=0,
            vmem_limit_bytes=64 * 1024 * 1024,
        ),
    )(order, x, w_mat)
